# baseline (speedup 1.0000x reference)
import functools

import jax
import jax.numpy as jnp
from jax import lax
from jax.experimental import pallas as pl
from jax.experimental.pallas import tpu as pltpu
from jax.experimental.pallas import tpu_sc as plsc

NUM_FIELDS = 26
BATCH = 16384
EMB_DIM = 16
NUM_WORKERS = 32
BPW = BATCH // NUM_WORKERS


def _make_sc_kernel():
    mesh = plsc.VectorSubcoreMesh(core_axis_name="c", subcore_axis_name="s")

    @functools.partial(
        pl.kernel,
        mesh=mesh,
        out_type=jax.ShapeDtypeStruct((BATCH, NUM_FIELDS * EMB_DIM), jnp.float32),
        scratch_types=[
            pltpu.VMEM((BPW,), jnp.int32),
            pltpu.VMEM((BPW, EMB_DIM), jnp.float32),
            pltpu.SemaphoreType.DMA,
        ],
        compiler_params=pltpu.CompilerParams(use_tc_tiling_on_sc=False),
    )
    def sc_embed(*refs):
        ins = refs[:2 * NUM_FIELDS]
        out = refs[2 * NUM_FIELDS]
        idx_v, rows_v, sem = refs[2 * NUM_FIELDS + 1:]
        wid = lax.axis_index("s") * 2 + lax.axis_index("c")
        base = wid * BPW
        for i in range(NUM_FIELDS):
            f_hbm = ins[2 * i]
            w_hbm = ins[2 * i + 1]
            pltpu.sync_copy(f_hbm.at[pl.ds(base, BPW)], idx_v)
            pltpu.async_copy(w_hbm.at[idx_v], rows_v, sem).wait()
            pltpu.sync_copy(
                rows_v, out.at[pl.ds(base, BPW), pl.ds(i * EMB_DIM, EMB_DIM)]
            )

    return sc_embed


_sc_embed = _make_sc_kernel()


def kernel(f0, W0, f1, W1, f2, W2, f3, W3, f4, W4, f5, W5, f6, W6, f7, W7,
           f8, W8, f9, W9, f10, W10, f11, W11, f12, W12, f13, W13, f14, W14,
           f15, W15, f16, W16, f17, W17, f18, W18, f19, W19, f20, W20,
           f21, W21, f22, W22, f23, W23, f24, W24, f25, W25):
    return _sc_embed(
        f0, W0, f1, W1, f2, W2, f3, W3, f4, W4, f5, W5, f6, W6, f7, W7,
        f8, W8, f9, W9, f10, W10, f11, W11, f12, W12, f13, W13, f14, W14,
        f15, W15, f16, W16, f17, W17, f18, W18, f19, W19, f20, W20,
        f21, W21, f22, W22, f23, W23, f24, W24, f25, W25)
